# Initial kernel scaffold; baseline (speedup 1.0000x reference)
import jax
import jax.numpy as jnp
from jax.experimental import pallas as pl


def kernel(x, edge_index, params):
    raise NotImplementedError("write your pallas kernel here")


# speedup vs baseline: 2.8078x; 2.8078x over previous
import functools
import math

import jax
import jax.numpy as jnp
from jax import lax
from jax.experimental import pallas as pl
from jax.experimental.pallas import tpu as pltpu
from jax.experimental.pallas import tpu_sc as plsc

N = 10000
E = 160000
H = 128
HEADS = 4
D_HEAD = H // HEADS

SC_CORES = 2
SC_SUBCORES = 16
SC_WORKERS = SC_CORES * SC_SUBCORES
CHUNK = 64
NCHUNKS = 80
EPAD = SC_WORKERS * NCHUNKS * CHUNK
NPAD = 10240
ROWS_PER_SUB = NPAD // SC_SUBCORES
HALF = NPAD // SC_CORES
ACC_ROWS = 5248
AGG_ROWS_PER_SUB = ACC_ROWS // SC_SUBCORES
AGG_CHUNKS_PER_SUB = EPAD // (SC_SUBCORES * CHUNK)


def _sc_mesh():
    return plsc.VectorSubcoreMesh(core_axis_name="c", subcore_axis_name="s")


def _sc_edge_aggregate(xs, srcp, dstp, zeros_ah):
    @functools.partial(
        pl.kernel,
        out_type=jax.ShapeDtypeStruct((SC_CORES, ACC_ROWS, H), jnp.float32),
        mesh=_sc_mesh(),
        scratch_types=[
            pltpu.VMEM((CHUNK, H), jnp.float32),
            pltpu.VMEM((1, CHUNK), jnp.int32),
            pltpu.VMEM((1, CHUNK), jnp.int32),
            pltpu.VMEM_SHARED((ACC_ROWS, H), jnp.float32),
            pltpu.SemaphoreType.DMA,
        ],
    )
    def agg_kernel(xs_hbm, src_hbm, dst_hbm, z_hbm, out_hbm, buf_v, isrc_v,
                   idst_v, acc_sh, sem):
        c = lax.axis_index("c")
        s = lax.axis_index("s")

        arow = s * AGG_ROWS_PER_SUB
        pltpu.sync_copy(z_hbm, acc_sh.at[pl.ds(arow, AGG_ROWS_PER_SUB)])
        plsc.subcore_barrier()

        base = s * AGG_CHUNKS_PER_SUB
        lo = c * HALF

        @pl.loop(0, AGG_CHUNKS_PER_SUB)
        def _(i):
            pltpu.sync_copy(src_hbm.at[pl.ds(base + i, 1)], isrc_v)
            pltpu.sync_copy(dst_hbm.at[pl.ds(base + i, 1)], idst_v)

            for j in range(0, CHUNK, 16):
                v = idst_v[0, pl.ds(j, 16)] - lo
                ok = (v >= 0) & (v < HALF)
                idst_v[0, pl.ds(j, 16)] = jnp.where(ok, v, HALF)

            pltpu.async_copy(xs_hbm.at[isrc_v.at[0]], buf_v, sem).wait()
            pltpu.sync_copy(buf_v, acc_sh.at[idst_v.at[0]], add=True)

        plsc.subcore_barrier()
        pltpu.sync_copy(acc_sh.at[pl.ds(arow, AGG_ROWS_PER_SUB)],
                        out_hbm.at[c].at[pl.ds(arow, AGG_ROWS_PER_SUB)])

    return agg_kernel(xs, srcp, dstp, zeros_ah)


_F32 = jnp.float32


def _mm_bias(x, w, b, relu, block_rows=2000):
    n, k = x.shape
    m = w.shape[1]

    def body(x_ref, w_ref, b_ref, o_ref):
        acc = jnp.dot(x_ref[...], w_ref[...], preferred_element_type=_F32)
        acc = acc + b_ref[...]
        if relu:
            acc = jnp.maximum(acc, 0.0)
        o_ref[...] = acc

    return pl.pallas_call(
        body,
        grid=(n // block_rows,),
        in_specs=[
            pl.BlockSpec((block_rows, k), lambda i: (i, 0)),
            pl.BlockSpec((k, m), lambda i: (0, 0)),
            pl.BlockSpec((1, m), lambda i: (0, 0)),
        ],
        out_specs=pl.BlockSpec((block_rows, m), lambda i: (i, 0)),
        out_shape=jax.ShapeDtypeStruct((n, m), _F32),
    )(x, w, b.reshape(1, m))


def _xs_kernel(h, w, degp, block_rows=2000):
    n = h.shape[0]

    def body(h_ref, w_ref, d_ref, o_ref):
        deg = 1.0 + d_ref[...][:, 0:1]
        dinv = lax.rsqrt(deg)
        xw = jnp.dot(h_ref[...], w_ref[...], preferred_element_type=_F32)
        o_ref[...] = xw * dinv

    return pl.pallas_call(
        body,
        grid=(n // block_rows,),
        in_specs=[
            pl.BlockSpec((block_rows, H), lambda i: (i, 0)),
            pl.BlockSpec((H, H), lambda i: (0, 0)),
            pl.BlockSpec((block_rows, H), lambda i: (i, 0)),
        ],
        out_specs=pl.BlockSpec((block_rows, H), lambda i: (i, 0)),
        out_shape=jax.ShapeDtypeStruct((n, H), _F32),
    )(h, w, degp)


def _attention_kernel(qkv, block_q=400):
    n = qkv.shape[0]
    scale = 1.0 / math.sqrt(D_HEAD)

    def body(q_ref, k_ref, v_ref, o_ref):
        q = q_ref[...] * scale
        k = k_ref[...]
        v = v_ref[...]
        head_of_col = lax.broadcasted_iota(jnp.int32, (1, H), 1) // D_HEAD
        acc = jnp.zeros((block_q, H), _F32)
        for hh in range(HEADS):
            qh = jnp.where(head_of_col == hh, q, 0.0)
            s = lax.dot_general(qh, k, (((1,), (1,)), ((), ())),
                                preferred_element_type=_F32)
            s = s - jnp.max(s, axis=1, keepdims=True)
            e = jnp.exp(s)
            attn = e / jnp.sum(e, axis=1, keepdims=True)
            av = jnp.dot(attn, v, preferred_element_type=_F32)
            acc = acc + jnp.where(head_of_col == hh, av, 0.0)
        o_ref[...] = acc

    return pl.pallas_call(
        body,
        grid=(n // block_q,),
        in_specs=[
            pl.BlockSpec((block_q, H), lambda i: (i, 0)),
            pl.BlockSpec((n, H), lambda i: (0, 1)),
            pl.BlockSpec((n, H), lambda i: (0, 2)),
        ],
        out_specs=pl.BlockSpec((block_q, H), lambda i: (i, 0)),
        out_shape=jax.ShapeDtypeStruct((n, H), _F32),
    )(qkv, qkv, qkv)


def _bn(v, g, b):
    mu = jnp.mean(v, axis=0, keepdims=True)
    var = jnp.mean((v - mu) * (v - mu), axis=0, keepdims=True)
    return (v - mu) * lax.rsqrt(var + 1e-5) * g + b


def _local_tail_kernel(h, xs, p_all, degp, gcn_b, bn1_g, bn1_b):
    def body(h_ref, xs_ref, p_ref, d_ref, gb_ref, g_ref, b_ref, o_ref):
        deg = 1.0 + d_ref[...][:, 0:1]
        dinv = lax.rsqrt(deg)
        gl = dinv * (p_ref[...] + xs_ref[...]) + gb_ref[...]
        o_ref[...] = _bn(gl + h_ref[...], g_ref[...], b_ref[...])

    return pl.pallas_call(
        body,
        out_shape=jax.ShapeDtypeStruct((N, H), _F32),
    )(h, xs, p_all, degp, gcn_b.reshape(1, H),
      bn1_g.reshape(1, H), bn1_b.reshape(1, H))


def _attn_tail_kernel(h, o, wo, bo, bn2_g, bn2_b):
    def body(h_ref, o_ref, wo_ref, bo_ref, g_ref, b_ref, out_ref):
        att = jnp.dot(o_ref[...], wo_ref[...], preferred_element_type=_F32)
        att = att + bo_ref[...] + h_ref[...]
        out_ref[...] = _bn(att, g_ref[...], b_ref[...])

    return pl.pallas_call(
        body,
        out_shape=jax.ShapeDtypeStruct((N, H), _F32),
    )(h, o, wo, bo.reshape(1, H), bn2_g.reshape(1, H), bn2_b.reshape(1, H))


def _ffn_tail_kernel(h_local, h_attn, w1, b1, w2, b2, bn3_g, bn3_b):
    hid = w1.shape[1]

    def body(hl_ref, ha_ref, w1_ref, b1_ref, w2_ref, b2_ref, g_ref, b_ref,
             out_ref):
        out = hl_ref[...] + ha_ref[...]
        f1 = jnp.maximum(
            jnp.dot(out, w1_ref[...], preferred_element_type=_F32)
            + b1_ref[...], 0.0)
        ff = jnp.dot(f1, w2_ref[...], preferred_element_type=_F32) + b2_ref[...]
        out_ref[...] = jnp.maximum(_bn(out + ff, g_ref[...], b_ref[...]), 0.0)

    return pl.pallas_call(
        body,
        out_shape=jax.ShapeDtypeStruct((N, H), _F32),
    )(h_local, h_attn, w1, b1.reshape(1, hid), w2, b2.reshape(1, H),
      bn3_g.reshape(1, H), bn3_b.reshape(1, H))


def _head_kernel(h, w1, b1, w2, b2):
    hid = w1.shape[1]
    dout = w2.shape[1]

    def body(h_ref, w1_ref, b1_ref, w2_ref, b2_ref, out_ref):
        f1 = jnp.maximum(
            jnp.dot(h_ref[...], w1_ref[...], preferred_element_type=_F32)
            + b1_ref[...], 0.0)
        out_ref[...] = (jnp.dot(f1, w2_ref[...], preferred_element_type=_F32)
                        + b2_ref[...])

    return pl.pallas_call(
        body,
        out_shape=jax.ShapeDtypeStruct((N, dout), _F32),
    )(h, w1, b1.reshape(1, hid), w2, b2.reshape(1, dout))


def _gps_layer(h, srcp, dstp, degp, p):
    xs = _xs_kernel(h, p['gcn_W'], degp)
    part = _sc_edge_aggregate(xs, srcp, dstp,
                              jnp.zeros((AGG_ROWS_PER_SUB, H), _F32))
    p_all = jnp.concatenate([part[0, :HALF, :], part[1, :N - HALF, :]], axis=0)

    wqkv = jnp.concatenate([p['Wq'], p['Wk'], p['Wv']], axis=1)
    bqkv = jnp.concatenate([p['bq'], p['bk'], p['bv']])
    qkv = _mm_bias(h, wqkv, bqkv, relu=False)
    o = _attention_kernel(qkv)

    h_local = _local_tail_kernel(h, xs, p_all, degp, p['gcn_b'], p['bn1_g'],
                                 p['bn1_b'])
    h_attn = _attn_tail_kernel(h, o, p['Wo'], p['bo'], p['bn2_g'], p['bn2_b'])
    return _ffn_tail_kernel(h_local, h_attn, p['ff_W1'], p['ff_b1'],
                            p['ff_W2'], p['ff_b2'], p['bn3_g'], p['bn3_b'])


@jax.jit
def _forward_impl(x, edge_index, params):
    src = edge_index[0]
    dst = edge_index[1]
    pad = EPAD - E
    srcp = jnp.concatenate([src, jnp.zeros((pad,), jnp.int32)])
    dstp = jnp.concatenate([dst, jnp.full((pad,), N, jnp.int32)])
    srcp = srcp.reshape(EPAD // CHUNK, CHUNK)
    dstp = dstp.reshape(EPAD // CHUNK, CHUNK)

    zeros_ah = jnp.zeros((AGG_ROWS_PER_SUB, H), _F32)
    degp_raw = _sc_edge_aggregate(jnp.ones((N, H), _F32), srcp, dstp, zeros_ah)
    degp = jnp.concatenate(
        [degp_raw[0, :HALF, :], degp_raw[1, :N - HALF, :]], axis=0)
    h = _mm_bias(x, params['pre_W'], params['pre_b'], relu=True)
    h = _gps_layer(h, srcp, dstp, degp, params['l1'])
    h = _gps_layer(h, srcp, dstp, degp, params['l2'])
    return _head_kernel(h, params['head_W1'], params['head_b1'],
                        params['head_W2'], params['head_b2'])


def kernel(x, edge_index, params):
    return _forward_impl(x, edge_index, params)


# speedup vs baseline: 2.9673x; 1.0568x over previous
import functools
import math

import jax
import jax.numpy as jnp
from jax import lax
from jax.experimental import pallas as pl
from jax.experimental.pallas import tpu as pltpu
from jax.experimental.pallas import tpu_sc as plsc

N = 10000
E = 160000
H = 128
HEADS = 4
D_HEAD = H // HEADS

SC_CORES = 2
SC_SUBCORES = 16
SC_WORKERS = SC_CORES * SC_SUBCORES
CHUNK = 128
EPAD = 163840
NPAD = 10240
ROWS_PER_SUB = NPAD // SC_SUBCORES
HALF = NPAD // SC_CORES
ACC_ROWS = 5248
AGG_ROWS_PER_SUB = ACC_ROWS // SC_SUBCORES
AGG_CHUNKS_PER_SUB = EPAD // (SC_SUBCORES * CHUNK)


def _sc_mesh():
    return plsc.VectorSubcoreMesh(core_axis_name="c", subcore_axis_name="s")


def _sc_edge_aggregate(xs, srcp, dstloc, zeros_ah):
    nch = EPAD // (SC_SUBCORES * CHUNK)

    @functools.partial(
        pl.kernel,
        out_type=jax.ShapeDtypeStruct((SC_CORES, ACC_ROWS, H), jnp.float32),
        mesh=_sc_mesh(),
        scratch_types=[
            pltpu.VMEM((2, CHUNK, H), jnp.float32),
            pltpu.VMEM((2, CHUNK), jnp.int32),
            pltpu.VMEM((2, CHUNK), jnp.int32),
            pltpu.VMEM_SHARED((ACC_ROWS, H), jnp.float32),
            pltpu.SemaphoreType.DMA,
            pltpu.SemaphoreType.DMA,
        ],
    )
    def agg_kernel(xs_hbm, src_hbm, dst_hbm, z_hbm, out_hbm, buf_v, isrc_v,
                   idst_v, acc_sh, sem0, sem1):
        c = lax.axis_index("c")
        s = lax.axis_index("s")

        arow = s * AGG_ROWS_PER_SUB
        pltpu.sync_copy(z_hbm, acc_sh.at[pl.ds(arow, AGG_ROWS_PER_SUB)])
        plsc.subcore_barrier()

        base = s * nch

        def load_idx(chunk, slot):
            pltpu.sync_copy(src_hbm.at[pl.ds(chunk, 1)],
                            isrc_v.at[pl.ds(slot, 1)])
            pltpu.sync_copy(dst_hbm.at[c].at[pl.ds(chunk, 1)],
                            idst_v.at[pl.ds(slot, 1)])

        load_idx(base, 0)
        pltpu.async_copy(xs_hbm.at[isrc_v.at[0]], buf_v.at[0], sem0)

        @pl.loop(0, nch, step=2)
        def _(i):
            load_idx(base + i + 1, 1)
            pltpu.async_copy(xs_hbm.at[isrc_v.at[1]], buf_v.at[1], sem1)
            pltpu.make_async_copy(xs_hbm.at[isrc_v.at[0]], buf_v.at[0],
                                  sem0).wait()
            pltpu.sync_copy(buf_v.at[0], acc_sh.at[idst_v.at[0]], add=True)

            @pl.when(i + 2 < nch)
            def _():
                load_idx(base + i + 2, 0)
                pltpu.async_copy(xs_hbm.at[isrc_v.at[0]], buf_v.at[0], sem0)

            pltpu.make_async_copy(xs_hbm.at[isrc_v.at[1]], buf_v.at[1],
                                  sem1).wait()
            pltpu.sync_copy(buf_v.at[1], acc_sh.at[idst_v.at[1]], add=True)

        plsc.subcore_barrier()
        pltpu.sync_copy(acc_sh.at[pl.ds(arow, AGG_ROWS_PER_SUB)],
                        out_hbm.at[c].at[pl.ds(arow, AGG_ROWS_PER_SUB)])

    return agg_kernel(xs, srcp, dstloc, zeros_ah)


_F32 = jnp.float32


def _mm_bias(x, w, b, relu, block_rows=2000):
    n, k = x.shape
    m = w.shape[1]

    def body(x_ref, w_ref, b_ref, o_ref):
        acc = jnp.dot(x_ref[...], w_ref[...], preferred_element_type=_F32)
        acc = acc + b_ref[...]
        if relu:
            acc = jnp.maximum(acc, 0.0)
        o_ref[...] = acc

    return pl.pallas_call(
        body,
        grid=(n // block_rows,),
        in_specs=[
            pl.BlockSpec((block_rows, k), lambda i: (i, 0)),
            pl.BlockSpec((k, m), lambda i: (0, 0)),
            pl.BlockSpec((1, m), lambda i: (0, 0)),
        ],
        out_specs=pl.BlockSpec((block_rows, m), lambda i: (i, 0)),
        out_shape=jax.ShapeDtypeStruct((n, m), _F32),
    )(x, w, b.reshape(1, m))


def _xs_kernel(h, w, degp, block_rows=2000):
    n = h.shape[0]

    def body(h_ref, w_ref, d_ref, o_ref):
        deg = 1.0 + d_ref[...][:, 0:1]
        dinv = lax.rsqrt(deg)
        xw = jnp.dot(h_ref[...], w_ref[...], preferred_element_type=_F32)
        o_ref[...] = xw * dinv

    return pl.pallas_call(
        body,
        grid=(n // block_rows,),
        in_specs=[
            pl.BlockSpec((block_rows, H), lambda i: (i, 0)),
            pl.BlockSpec((H, H), lambda i: (0, 0)),
            pl.BlockSpec((block_rows, H), lambda i: (i, 0)),
        ],
        out_specs=pl.BlockSpec((block_rows, H), lambda i: (i, 0)),
        out_shape=jax.ShapeDtypeStruct((n, H), _F32),
    )(h, w, degp)


def _attention_kernel(qkv, block_q=400):
    n = qkv.shape[0]
    scale = 1.0 / math.sqrt(D_HEAD)

    def body(q_ref, k_ref, v_ref, o_ref):
        q = q_ref[...] * scale
        k = k_ref[...]
        v = v_ref[...]
        head_of_col = lax.broadcasted_iota(jnp.int32, (1, H), 1) // D_HEAD
        acc = jnp.zeros((block_q, H), _F32)
        for hh in range(HEADS):
            qh = jnp.where(head_of_col == hh, q, 0.0)
            s = lax.dot_general(qh, k, (((1,), (1,)), ((), ())),
                                preferred_element_type=_F32)
            s = s - jnp.max(s, axis=1, keepdims=True)
            e = jnp.exp(s)
            attn = e / jnp.sum(e, axis=1, keepdims=True)
            av = jnp.dot(attn, v, preferred_element_type=_F32)
            acc = acc + jnp.where(head_of_col == hh, av, 0.0)
        o_ref[...] = acc

    return pl.pallas_call(
        body,
        grid=(n // block_q,),
        in_specs=[
            pl.BlockSpec((block_q, H), lambda i: (i, 0)),
            pl.BlockSpec((n, H), lambda i: (0, 1)),
            pl.BlockSpec((n, H), lambda i: (0, 2)),
        ],
        out_specs=pl.BlockSpec((block_q, H), lambda i: (i, 0)),
        out_shape=jax.ShapeDtypeStruct((n, H), _F32),
    )(qkv, qkv, qkv)


def _bn(v, g, b):
    mu = jnp.mean(v, axis=0, keepdims=True)
    var = jnp.mean((v - mu) * (v - mu), axis=0, keepdims=True)
    return (v - mu) * lax.rsqrt(var + 1e-5) * g + b


def _local_tail_kernel(h, xs, p_all, degp, gcn_b, bn1_g, bn1_b):
    def body(h_ref, xs_ref, p_ref, d_ref, gb_ref, g_ref, b_ref, o_ref):
        deg = 1.0 + d_ref[...][:, 0:1]
        dinv = lax.rsqrt(deg)
        gl = dinv * (p_ref[...] + xs_ref[...]) + gb_ref[...]
        o_ref[...] = _bn(gl + h_ref[...], g_ref[...], b_ref[...])

    return pl.pallas_call(
        body,
        out_shape=jax.ShapeDtypeStruct((N, H), _F32),
    )(h, xs, p_all, degp, gcn_b.reshape(1, H),
      bn1_g.reshape(1, H), bn1_b.reshape(1, H))


def _attn_tail_kernel(h, o, wo, bo, bn2_g, bn2_b):
    def body(h_ref, o_ref, wo_ref, bo_ref, g_ref, b_ref, out_ref):
        att = jnp.dot(o_ref[...], wo_ref[...], preferred_element_type=_F32)
        att = att + bo_ref[...] + h_ref[...]
        out_ref[...] = _bn(att, g_ref[...], b_ref[...])

    return pl.pallas_call(
        body,
        out_shape=jax.ShapeDtypeStruct((N, H), _F32),
    )(h, o, wo, bo.reshape(1, H), bn2_g.reshape(1, H), bn2_b.reshape(1, H))


def _ffn_tail_kernel(h_local, h_attn, w1, b1, w2, b2, bn3_g, bn3_b):
    hid = w1.shape[1]

    def body(hl_ref, ha_ref, w1_ref, b1_ref, w2_ref, b2_ref, g_ref, b_ref,
             out_ref):
        out = hl_ref[...] + ha_ref[...]
        f1 = jnp.maximum(
            jnp.dot(out, w1_ref[...], preferred_element_type=_F32)
            + b1_ref[...], 0.0)
        ff = jnp.dot(f1, w2_ref[...], preferred_element_type=_F32) + b2_ref[...]
        out_ref[...] = jnp.maximum(_bn(out + ff, g_ref[...], b_ref[...]), 0.0)

    return pl.pallas_call(
        body,
        out_shape=jax.ShapeDtypeStruct((N, H), _F32),
    )(h_local, h_attn, w1, b1.reshape(1, hid), w2, b2.reshape(1, H),
      bn3_g.reshape(1, H), bn3_b.reshape(1, H))


def _head_kernel(h, w1, b1, w2, b2):
    hid = w1.shape[1]
    dout = w2.shape[1]

    def body(h_ref, w1_ref, b1_ref, w2_ref, b2_ref, out_ref):
        f1 = jnp.maximum(
            jnp.dot(h_ref[...], w1_ref[...], preferred_element_type=_F32)
            + b1_ref[...], 0.0)
        out_ref[...] = (jnp.dot(f1, w2_ref[...], preferred_element_type=_F32)
                        + b2_ref[...])

    return pl.pallas_call(
        body,
        out_shape=jax.ShapeDtypeStruct((N, dout), _F32),
    )(h, w1, b1.reshape(1, hid), w2, b2.reshape(1, dout))


def _gps_layer(h, srcp, dstloc, degp, p):
    xs = _xs_kernel(h, p['gcn_W'], degp)
    part = _sc_edge_aggregate(xs, srcp, dstloc,
                              jnp.zeros((AGG_ROWS_PER_SUB, H), _F32))
    p_all = jnp.concatenate([part[0, :HALF, :], part[1, :N - HALF, :]], axis=0)

    wqkv = jnp.concatenate([p['Wq'], p['Wk'], p['Wv']], axis=1)
    bqkv = jnp.concatenate([p['bq'], p['bk'], p['bv']])
    qkv = _mm_bias(h, wqkv, bqkv, relu=False)
    o = _attention_kernel(qkv)

    h_local = _local_tail_kernel(h, xs, p_all, degp, p['gcn_b'], p['bn1_g'],
                                 p['bn1_b'])
    h_attn = _attn_tail_kernel(h, o, p['Wo'], p['bo'], p['bn2_g'], p['bn2_b'])
    return _ffn_tail_kernel(h_local, h_attn, p['ff_W1'], p['ff_b1'],
                            p['ff_W2'], p['ff_b2'], p['bn3_g'], p['bn3_b'])


@jax.jit
def _forward_impl(x, edge_index, params):
    src = edge_index[0]
    dst = edge_index[1]
    pad = EPAD - E
    srcp = jnp.concatenate([src, jnp.zeros((pad,), jnp.int32)])
    dstp = jnp.concatenate([dst, jnp.full((pad,), N, jnp.int32)])
    srcp = srcp.reshape(EPAD // CHUNK, CHUNK)
    los = jnp.array([0, HALF], jnp.int32)[:, None]
    dl = dstp[None, :] - los
    dstloc = jnp.where((dl >= 0) & (dl < HALF), dl, HALF).astype(jnp.int32)
    dstloc = dstloc.reshape(SC_CORES, EPAD // CHUNK, CHUNK)

    zeros_ah = jnp.zeros((AGG_ROWS_PER_SUB, H), _F32)
    degp_raw = _sc_edge_aggregate(jnp.ones((N, H), _F32), srcp, dstloc,
                                  zeros_ah)
    degp = jnp.concatenate(
        [degp_raw[0, :HALF, :], degp_raw[1, :N - HALF, :]], axis=0)
    h = _mm_bias(x, params['pre_W'], params['pre_b'], relu=True)
    h = _gps_layer(h, srcp, dstloc, degp, params['l1'])
    h = _gps_layer(h, srcp, dstloc, degp, params['l2'])
    return _head_kernel(h, params['head_W1'], params['head_b1'],
                        params['head_W2'], params['head_b2'])


def kernel(x, edge_index, params):
    return _forward_impl(x, edge_index, params)


# speedup vs baseline: 4.3585x; 1.4688x over previous
import functools
import math

import jax
import jax.numpy as jnp
from jax import lax
from jax.experimental import pallas as pl
from jax.experimental.pallas import tpu as pltpu
from jax.experimental.pallas import tpu_sc as plsc

N = 10000
E = 160000
H = 128
HEADS = 4
D_HEAD = H // HEADS

SC_CORES = 2
SC_SUBCORES = 16
SC_WORKERS = SC_CORES * SC_SUBCORES
CHUNK = 128
EPAD = 163840
NPAD = 10240
ROWS_PER_SUB = NPAD // SC_SUBCORES
HALF = NPAD // SC_CORES
ACC_ROWS = 5248
AGG_ROWS_PER_SUB = ACC_ROWS // SC_SUBCORES
AGG_CHUNKS_PER_SUB = EPAD // (SC_SUBCORES * CHUNK)


def _sc_mesh():
    return plsc.VectorSubcoreMesh(core_axis_name="c", subcore_axis_name="s")


def _sc_edge_aggregate(xs, srcp, dstloc, zeros_ah):
    nch = EPAD // (SC_SUBCORES * CHUNK)

    @functools.partial(
        pl.kernel,
        out_type=jax.ShapeDtypeStruct((SC_CORES, ACC_ROWS, H), jnp.float32),
        mesh=_sc_mesh(),
        scratch_types=[
            pltpu.VMEM((2, CHUNK, H), jnp.float32),
            pltpu.VMEM((2, CHUNK), jnp.int32),
            pltpu.VMEM((2, CHUNK), jnp.int32),
            pltpu.VMEM_SHARED((ACC_ROWS, H), jnp.float32),
            pltpu.SemaphoreType.DMA,
            pltpu.SemaphoreType.DMA,
        ],
    )
    def agg_kernel(xs_hbm, src_hbm, dst_hbm, z_hbm, out_hbm, buf_v, isrc_v,
                   idst_v, acc_sh, sem0, sem1):
        c = lax.axis_index("c")
        s = lax.axis_index("s")

        arow = s * AGG_ROWS_PER_SUB
        pltpu.sync_copy(z_hbm, acc_sh.at[pl.ds(arow, AGG_ROWS_PER_SUB)])
        plsc.subcore_barrier()

        base = s * nch

        def load_idx(chunk, slot):
            pltpu.sync_copy(src_hbm.at[pl.ds(chunk, 1)],
                            isrc_v.at[pl.ds(slot, 1)])
            pltpu.sync_copy(dst_hbm.at[c].at[pl.ds(chunk, 1)],
                            idst_v.at[pl.ds(slot, 1)])

        load_idx(base, 0)
        pltpu.async_copy(xs_hbm.at[isrc_v.at[0]], buf_v.at[0], sem0)

        @pl.loop(0, nch, step=2)
        def _(i):
            load_idx(base + i + 1, 1)
            pltpu.async_copy(xs_hbm.at[isrc_v.at[1]], buf_v.at[1], sem1)
            pltpu.make_async_copy(xs_hbm.at[isrc_v.at[0]], buf_v.at[0],
                                  sem0).wait()
            pltpu.sync_copy(buf_v.at[0], acc_sh.at[idst_v.at[0]], add=True)

            @pl.when(i + 2 < nch)
            def _():
                load_idx(base + i + 2, 0)
                pltpu.async_copy(xs_hbm.at[isrc_v.at[0]], buf_v.at[0], sem0)

            pltpu.make_async_copy(xs_hbm.at[isrc_v.at[1]], buf_v.at[1],
                                  sem1).wait()
            pltpu.sync_copy(buf_v.at[1], acc_sh.at[idst_v.at[1]], add=True)

        plsc.subcore_barrier()
        pltpu.sync_copy(acc_sh.at[pl.ds(arow, AGG_ROWS_PER_SUB)],
                        out_hbm.at[c].at[pl.ds(arow, AGG_ROWS_PER_SUB)])

    return agg_kernel(xs, srcp, dstloc, zeros_ah)


_F32 = jnp.float32


def _mm_bias(x, w, b, relu, block_rows=2000):
    n, k = x.shape
    m = w.shape[1]

    def body(x_ref, w_ref, b_ref, o_ref):
        acc = jnp.dot(x_ref[...], w_ref[...], preferred_element_type=_F32)
        acc = acc + b_ref[...]
        if relu:
            acc = jnp.maximum(acc, 0.0)
        o_ref[...] = acc

    return pl.pallas_call(
        body,
        grid=(n // block_rows,),
        in_specs=[
            pl.BlockSpec((block_rows, k), lambda i: (i, 0)),
            pl.BlockSpec((k, m), lambda i: (0, 0)),
            pl.BlockSpec((1, m), lambda i: (0, 0)),
        ],
        out_specs=pl.BlockSpec((block_rows, m), lambda i: (i, 0)),
        out_shape=jax.ShapeDtypeStruct((n, m), _F32),
    )(x, w, b.reshape(1, m))


def _xs_kernel(h, w, degp, block_rows=2000):
    n = h.shape[0]

    def body(h_ref, w_ref, d_ref, o_ref):
        deg = 1.0 + d_ref[...][:, 0:1]
        dinv = lax.rsqrt(deg)
        xw = jnp.dot(h_ref[...], w_ref[...], preferred_element_type=_F32)
        o_ref[...] = xw * dinv

    return pl.pallas_call(
        body,
        grid=(n // block_rows,),
        in_specs=[
            pl.BlockSpec((block_rows, H), lambda i: (i, 0)),
            pl.BlockSpec((H, H), lambda i: (0, 0)),
            pl.BlockSpec((block_rows, H), lambda i: (i, 0)),
        ],
        out_specs=pl.BlockSpec((block_rows, H), lambda i: (i, 0)),
        out_shape=jax.ShapeDtypeStruct((n, H), _F32),
    )(h, w, degp)


def _attention_kernel(qkv, block_q=400):
    n = qkv.shape[0]
    scale = 1.0 / math.sqrt(D_HEAD)

    def body(q_ref, k_ref, v_ref, o_ref):
        q = q_ref[...] * scale
        k = k_ref[...]
        v = v_ref[...]
        head_of_col = lax.broadcasted_iota(jnp.int32, (1, H), 1) // D_HEAD
        acc = jnp.zeros((block_q, H), _F32)
        for hh in range(HEADS):
            qh = jnp.where(head_of_col == hh, q, 0.0)
            s = lax.dot_general(qh, k, (((1,), (1,)), ((), ())),
                                preferred_element_type=_F32)
            e = jnp.exp(s)
            den = jnp.sum(e, axis=1, keepdims=True)
            av = jnp.dot(e, v, preferred_element_type=_F32)
            acc = acc + jnp.where(head_of_col == hh, av / den, 0.0)
        o_ref[...] = acc

    return pl.pallas_call(
        body,
        grid=(n // block_q,),
        in_specs=[
            pl.BlockSpec((block_q, H), lambda i: (i, 0)),
            pl.BlockSpec((n, H), lambda i: (0, 1)),
            pl.BlockSpec((n, H), lambda i: (0, 2)),
        ],
        out_specs=pl.BlockSpec((block_q, H), lambda i: (i, 0)),
        out_shape=jax.ShapeDtypeStruct((n, H), _F32),
    )(qkv, qkv, qkv)


def _bn(v, g, b):
    mu = jnp.mean(v, axis=0, keepdims=True)
    var = jnp.mean((v - mu) * (v - mu), axis=0, keepdims=True)
    return (v - mu) * lax.rsqrt(var + 1e-5) * g + b


def _local_tail_kernel(h, xs, p_all, degp, gcn_b, bn1_g, bn1_b):
    def body(h_ref, xs_ref, p_ref, d_ref, gb_ref, g_ref, b_ref, o_ref):
        deg = 1.0 + d_ref[...][:, 0:1]
        dinv = lax.rsqrt(deg)
        gl = dinv * (p_ref[...] + xs_ref[...]) + gb_ref[...]
        o_ref[...] = _bn(gl + h_ref[...], g_ref[...], b_ref[...])

    return pl.pallas_call(
        body,
        out_shape=jax.ShapeDtypeStruct((N, H), _F32),
    )(h, xs, p_all, degp, gcn_b.reshape(1, H),
      bn1_g.reshape(1, H), bn1_b.reshape(1, H))


def _attn_tail_kernel(h, o, wo, bo, bn2_g, bn2_b):
    def body(h_ref, o_ref, wo_ref, bo_ref, g_ref, b_ref, out_ref):
        att = jnp.dot(o_ref[...], wo_ref[...], preferred_element_type=_F32)
        att = att + bo_ref[...] + h_ref[...]
        out_ref[...] = _bn(att, g_ref[...], b_ref[...])

    return pl.pallas_call(
        body,
        out_shape=jax.ShapeDtypeStruct((N, H), _F32),
    )(h, o, wo, bo.reshape(1, H), bn2_g.reshape(1, H), bn2_b.reshape(1, H))


def _ffn_tail_kernel(h_local, h_attn, w1, b1, w2, b2, bn3_g, bn3_b):
    hid = w1.shape[1]

    def body(hl_ref, ha_ref, w1_ref, b1_ref, w2_ref, b2_ref, g_ref, b_ref,
             out_ref):
        out = hl_ref[...] + ha_ref[...]
        f1 = jnp.maximum(
            jnp.dot(out, w1_ref[...], preferred_element_type=_F32)
            + b1_ref[...], 0.0)
        ff = jnp.dot(f1, w2_ref[...], preferred_element_type=_F32) + b2_ref[...]
        out_ref[...] = jnp.maximum(_bn(out + ff, g_ref[...], b_ref[...]), 0.0)

    return pl.pallas_call(
        body,
        out_shape=jax.ShapeDtypeStruct((N, H), _F32),
    )(h_local, h_attn, w1, b1.reshape(1, hid), w2, b2.reshape(1, H),
      bn3_g.reshape(1, H), bn3_b.reshape(1, H))


def _head_kernel(h, w1, b1, w2, b2):
    hid = w1.shape[1]
    dout = w2.shape[1]

    def body(h_ref, w1_ref, b1_ref, w2_ref, b2_ref, out_ref):
        f1 = jnp.maximum(
            jnp.dot(h_ref[...], w1_ref[...], preferred_element_type=_F32)
            + b1_ref[...], 0.0)
        out_ref[...] = (jnp.dot(f1, w2_ref[...], preferred_element_type=_F32)
                        + b2_ref[...])

    return pl.pallas_call(
        body,
        out_shape=jax.ShapeDtypeStruct((N, dout), _F32),
    )(h, w1, b1.reshape(1, hid), w2, b2.reshape(1, dout))


def _gps_layer(h, srcp, dstloc, degp, p):
    xs = _xs_kernel(h, p['gcn_W'], degp)
    part = _sc_edge_aggregate(xs, srcp, dstloc,
                              jnp.zeros((AGG_ROWS_PER_SUB, H), _F32))
    p_all = jnp.concatenate([part[0, :HALF, :], part[1, :N - HALF, :]], axis=0)

    wqkv = jnp.concatenate([p['Wq'], p['Wk'], p['Wv']], axis=1)
    bqkv = jnp.concatenate([p['bq'], p['bk'], p['bv']])
    qkv = _mm_bias(h, wqkv, bqkv, relu=False)
    o = _attention_kernel(qkv)

    h_local = _local_tail_kernel(h, xs, p_all, degp, p['gcn_b'], p['bn1_g'],
                                 p['bn1_b'])
    h_attn = _attn_tail_kernel(h, o, p['Wo'], p['bo'], p['bn2_g'], p['bn2_b'])
    return _ffn_tail_kernel(h_local, h_attn, p['ff_W1'], p['ff_b1'],
                            p['ff_W2'], p['ff_b2'], p['bn3_g'], p['bn3_b'])


@jax.jit
def _forward_impl(x, edge_index, params):
    src = edge_index[0]
    dst = edge_index[1]
    pad = EPAD - E
    srcp = jnp.concatenate([src, jnp.zeros((pad,), jnp.int32)])
    dstp = jnp.concatenate([dst, jnp.full((pad,), N, jnp.int32)])
    srcp = srcp.reshape(EPAD // CHUNK, CHUNK)
    los = jnp.array([0, HALF], jnp.int32)[:, None]
    dl = dstp[None, :] - los
    dstloc = jnp.where((dl >= 0) & (dl < HALF), dl, HALF).astype(jnp.int32)
    dstloc = dstloc.reshape(SC_CORES, EPAD // CHUNK, CHUNK)

    zeros_ah = jnp.zeros((AGG_ROWS_PER_SUB, H), _F32)
    degp_raw = _sc_edge_aggregate(jnp.ones((N, H), _F32), srcp, dstloc,
                                  zeros_ah)
    degp = jnp.concatenate(
        [degp_raw[0, :HALF, :], degp_raw[1, :N - HALF, :]], axis=0)
    h = _mm_bias(x, params['pre_W'], params['pre_b'], relu=True)
    h = _gps_layer(h, srcp, dstloc, degp, params['l1'])
    h = _gps_layer(h, srcp, dstloc, degp, params['l2'])
    return _head_kernel(h, params['head_W1'], params['head_b1'],
                        params['head_W2'], params['head_b2'])


def kernel(x, edge_index, params):
    return _forward_impl(x, edge_index, params)


# speedup vs baseline: 5.5430x; 1.2718x over previous
import functools
import math

import jax
import jax.numpy as jnp
from jax import lax
from jax.experimental import pallas as pl
from jax.experimental.pallas import tpu as pltpu
from jax.experimental.pallas import tpu_sc as plsc

N = 10000
E = 160000
H = 128
HEADS = 4
D_HEAD = H // HEADS

SC_CORES = 2
SC_SUBCORES = 16
SC_WORKERS = SC_CORES * SC_SUBCORES
CHUNK = 128
EPAD = 163840
NPAD = 10240
ROWS_PER_SUB = NPAD // SC_SUBCORES
HALF = NPAD // SC_CORES
ACC_ROWS = 5248
AGG_ROWS_PER_SUB = ACC_ROWS // SC_SUBCORES
AGG_CHUNKS_PER_SUB = EPAD // (SC_SUBCORES * CHUNK)


def _sc_mesh():
    return plsc.VectorSubcoreMesh(core_axis_name="c", subcore_axis_name="s")


def _sc_edge_aggregate(xs, srcp, dstloc, zeros_ah):
    nch = EPAD // (SC_SUBCORES * CHUNK)

    @functools.partial(
        pl.kernel,
        out_type=jax.ShapeDtypeStruct((SC_CORES, ACC_ROWS, H), jnp.float32),
        mesh=_sc_mesh(),
        scratch_types=[
            pltpu.VMEM((2, CHUNK, H), jnp.float32),
            pltpu.VMEM((2, CHUNK), jnp.int32),
            pltpu.VMEM((2, CHUNK), jnp.int32),
            pltpu.VMEM_SHARED((ACC_ROWS, H), jnp.float32),
            pltpu.SemaphoreType.DMA,
            pltpu.SemaphoreType.DMA,
        ],
    )
    def agg_kernel(xs_hbm, src_hbm, dst_hbm, z_hbm, out_hbm, buf_v, isrc_v,
                   idst_v, acc_sh, sem0, sem1):
        c = lax.axis_index("c")
        s = lax.axis_index("s")

        arow = s * AGG_ROWS_PER_SUB
        pltpu.sync_copy(z_hbm, acc_sh.at[pl.ds(arow, AGG_ROWS_PER_SUB)])
        plsc.subcore_barrier()

        base = s * nch

        def load_idx(chunk, slot):
            pltpu.sync_copy(src_hbm.at[pl.ds(chunk, 1)],
                            isrc_v.at[pl.ds(slot, 1)])
            pltpu.sync_copy(dst_hbm.at[c].at[pl.ds(chunk, 1)],
                            idst_v.at[pl.ds(slot, 1)])

        load_idx(base, 0)
        pltpu.async_copy(xs_hbm.at[isrc_v.at[0]], buf_v.at[0], sem0)

        @pl.loop(0, nch, step=2)
        def _(i):
            load_idx(base + i + 1, 1)
            pltpu.async_copy(xs_hbm.at[isrc_v.at[1]], buf_v.at[1], sem1)
            pltpu.make_async_copy(xs_hbm.at[isrc_v.at[0]], buf_v.at[0],
                                  sem0).wait()
            pltpu.sync_copy(buf_v.at[0], acc_sh.at[idst_v.at[0]], add=True)

            @pl.when(i + 2 < nch)
            def _():
                load_idx(base + i + 2, 0)
                pltpu.async_copy(xs_hbm.at[isrc_v.at[0]], buf_v.at[0], sem0)

            pltpu.make_async_copy(xs_hbm.at[isrc_v.at[1]], buf_v.at[1],
                                  sem1).wait()
            pltpu.sync_copy(buf_v.at[1], acc_sh.at[idst_v.at[1]], add=True)

        plsc.subcore_barrier()
        pltpu.sync_copy(acc_sh.at[pl.ds(arow, AGG_ROWS_PER_SUB)],
                        out_hbm.at[c].at[pl.ds(arow, AGG_ROWS_PER_SUB)])

    return agg_kernel(xs, srcp, dstloc, zeros_ah)


def _sc_edge_count(dstloc, ones_ch, zeros_ah):
    nch = EPAD // (SC_SUBCORES * CHUNK)

    @functools.partial(
        pl.kernel,
        out_type=jax.ShapeDtypeStruct((SC_CORES, ACC_ROWS, H), jnp.float32),
        mesh=_sc_mesh(),
        scratch_types=[
            pltpu.VMEM((CHUNK, H), jnp.float32),
            pltpu.VMEM((2, CHUNK), jnp.int32),
            pltpu.VMEM_SHARED((ACC_ROWS, H), jnp.float32),
            pltpu.SemaphoreType.DMA,
        ],
    )
    def count_kernel(dst_hbm, one_hbm, z_hbm, out_hbm, buf_v, idst_v, acc_sh,
                     sem):
        c = lax.axis_index("c")
        s = lax.axis_index("s")

        arow = s * AGG_ROWS_PER_SUB
        pltpu.sync_copy(z_hbm, acc_sh.at[pl.ds(arow, AGG_ROWS_PER_SUB)])
        pltpu.sync_copy(one_hbm, buf_v)
        plsc.subcore_barrier()

        base = s * nch

        pltpu.sync_copy(dst_hbm.at[c].at[pl.ds(base, 1)],
                        idst_v.at[pl.ds(0, 1)])

        @pl.loop(0, nch, step=2)
        def _(i):
            pltpu.sync_copy(dst_hbm.at[c].at[pl.ds(base + i + 1, 1)],
                            idst_v.at[pl.ds(1, 1)])
            pltpu.sync_copy(buf_v, acc_sh.at[idst_v.at[0]], add=True)

            @pl.when(i + 2 < nch)
            def _():
                pltpu.sync_copy(dst_hbm.at[c].at[pl.ds(base + i + 2, 1)],
                                idst_v.at[pl.ds(0, 1)])

            pltpu.sync_copy(buf_v, acc_sh.at[idst_v.at[1]], add=True)

        plsc.subcore_barrier()
        pltpu.sync_copy(acc_sh.at[pl.ds(arow, AGG_ROWS_PER_SUB)],
                        out_hbm.at[c].at[pl.ds(arow, AGG_ROWS_PER_SUB)])

    return count_kernel(dstloc, ones_ch, zeros_ah)


_F32 = jnp.float32


def _mm_bias(x, w, b, relu, block_rows=2000):
    n, k = x.shape
    m = w.shape[1]

    def body(x_ref, w_ref, b_ref, o_ref):
        acc = jnp.dot(x_ref[...], w_ref[...], preferred_element_type=_F32)
        acc = acc + b_ref[...]
        if relu:
            acc = jnp.maximum(acc, 0.0)
        o_ref[...] = acc

    return pl.pallas_call(
        body,
        grid=(n // block_rows,),
        in_specs=[
            pl.BlockSpec((block_rows, k), lambda i: (i, 0)),
            pl.BlockSpec((k, m), lambda i: (0, 0)),
            pl.BlockSpec((1, m), lambda i: (0, 0)),
        ],
        out_specs=pl.BlockSpec((block_rows, m), lambda i: (i, 0)),
        out_shape=jax.ShapeDtypeStruct((n, m), _F32),
    )(x, w, b.reshape(1, m))


def _xs_kernel(h, w, degp, block_rows=2000):
    n = h.shape[0]

    def body(h_ref, w_ref, d_ref, o_ref):
        deg = 1.0 + d_ref[...][:, 0:1]
        dinv = lax.rsqrt(deg)
        xw = jnp.dot(h_ref[...], w_ref[...], preferred_element_type=_F32)
        o_ref[...] = xw * dinv

    return pl.pallas_call(
        body,
        grid=(n // block_rows,),
        in_specs=[
            pl.BlockSpec((block_rows, H), lambda i: (i, 0)),
            pl.BlockSpec((H, H), lambda i: (0, 0)),
            pl.BlockSpec((block_rows, H), lambda i: (i, 0)),
        ],
        out_specs=pl.BlockSpec((block_rows, H), lambda i: (i, 0)),
        out_shape=jax.ShapeDtypeStruct((n, H), _F32),
    )(h, w, degp)


def _attention_kernel(qkv, block_q=400):
    n = qkv.shape[0]
    scale = 1.0 / math.sqrt(D_HEAD)

    def body(q_ref, k_ref, v_ref, o_ref):
        q = q_ref[...] * scale
        k = k_ref[...]
        v = v_ref[...]
        head_of_col = lax.broadcasted_iota(jnp.int32, (1, H), 1) // D_HEAD
        acc = jnp.zeros((block_q, H), _F32)
        for hh in range(HEADS):
            qh = jnp.where(head_of_col == hh, q, 0.0)
            s = lax.dot_general(qh, k, (((1,), (1,)), ((), ())),
                                preferred_element_type=_F32)
            e = jnp.exp(s)
            den = jnp.sum(e, axis=1, keepdims=True)
            av = jnp.dot(e, v, preferred_element_type=_F32)
            acc = acc + jnp.where(head_of_col == hh, av / den, 0.0)
        o_ref[...] = acc

    return pl.pallas_call(
        body,
        grid=(n // block_q,),
        in_specs=[
            pl.BlockSpec((block_q, H), lambda i: (i, 0)),
            pl.BlockSpec((n, H), lambda i: (0, 1)),
            pl.BlockSpec((n, H), lambda i: (0, 2)),
        ],
        out_specs=pl.BlockSpec((block_q, H), lambda i: (i, 0)),
        out_shape=jax.ShapeDtypeStruct((n, H), _F32),
    )(qkv, qkv, qkv)


def _bn(v, g, b):
    mu = jnp.mean(v, axis=0, keepdims=True)
    var = jnp.mean((v - mu) * (v - mu), axis=0, keepdims=True)
    return (v - mu) * lax.rsqrt(var + 1e-5) * g + b


def _local_tail_kernel(h, xs, p_all, degp, gcn_b, bn1_g, bn1_b):
    def body(h_ref, xs_ref, p_ref, d_ref, gb_ref, g_ref, b_ref, o_ref):
        deg = 1.0 + d_ref[...][:, 0:1]
        dinv = lax.rsqrt(deg)
        gl = dinv * (p_ref[...] + xs_ref[...]) + gb_ref[...]
        o_ref[...] = _bn(gl + h_ref[...], g_ref[...], b_ref[...])

    return pl.pallas_call(
        body,
        out_shape=jax.ShapeDtypeStruct((N, H), _F32),
    )(h, xs, p_all, degp, gcn_b.reshape(1, H),
      bn1_g.reshape(1, H), bn1_b.reshape(1, H))


def _attn_tail_kernel(h, o, wo, bo, bn2_g, bn2_b):
    def body(h_ref, o_ref, wo_ref, bo_ref, g_ref, b_ref, out_ref):
        att = jnp.dot(o_ref[...], wo_ref[...], preferred_element_type=_F32)
        att = att + bo_ref[...] + h_ref[...]
        out_ref[...] = _bn(att, g_ref[...], b_ref[...])

    return pl.pallas_call(
        body,
        out_shape=jax.ShapeDtypeStruct((N, H), _F32),
    )(h, o, wo, bo.reshape(1, H), bn2_g.reshape(1, H), bn2_b.reshape(1, H))


def _ffn_tail_kernel(h_local, h_attn, w1, b1, w2, b2, bn3_g, bn3_b):
    hid = w1.shape[1]

    def body(hl_ref, ha_ref, w1_ref, b1_ref, w2_ref, b2_ref, g_ref, b_ref,
             out_ref):
        out = hl_ref[...] + ha_ref[...]
        f1 = jnp.maximum(
            jnp.dot(out, w1_ref[...], preferred_element_type=_F32)
            + b1_ref[...], 0.0)
        ff = jnp.dot(f1, w2_ref[...], preferred_element_type=_F32) + b2_ref[...]
        out_ref[...] = jnp.maximum(_bn(out + ff, g_ref[...], b_ref[...]), 0.0)

    return pl.pallas_call(
        body,
        out_shape=jax.ShapeDtypeStruct((N, H), _F32),
    )(h_local, h_attn, w1, b1.reshape(1, hid), w2, b2.reshape(1, H),
      bn3_g.reshape(1, H), bn3_b.reshape(1, H))


def _head_kernel(h, w1, b1, w2, b2):
    hid = w1.shape[1]
    dout = w2.shape[1]

    def body(h_ref, w1_ref, b1_ref, w2_ref, b2_ref, out_ref):
        f1 = jnp.maximum(
            jnp.dot(h_ref[...], w1_ref[...], preferred_element_type=_F32)
            + b1_ref[...], 0.0)
        out_ref[...] = (jnp.dot(f1, w2_ref[...], preferred_element_type=_F32)
                        + b2_ref[...])

    return pl.pallas_call(
        body,
        out_shape=jax.ShapeDtypeStruct((N, dout), _F32),
    )(h, w1, b1.reshape(1, hid), w2, b2.reshape(1, dout))


def _gps_layer(h, srcp, dstloc, degp, p):
    xs = _xs_kernel(h, p['gcn_W'], degp)
    part = _sc_edge_aggregate(xs, srcp, dstloc,
                              jnp.zeros((AGG_ROWS_PER_SUB, H), _F32))
    p_all = jnp.concatenate([part[0, :HALF, :], part[1, :N - HALF, :]], axis=0)

    wqkv = jnp.concatenate([p['Wq'], p['Wk'], p['Wv']], axis=1)
    bqkv = jnp.concatenate([p['bq'], p['bk'], p['bv']])
    qkv = _mm_bias(h, wqkv, bqkv, relu=False)
    o = _attention_kernel(qkv)

    h_local = _local_tail_kernel(h, xs, p_all, degp, p['gcn_b'], p['bn1_g'],
                                 p['bn1_b'])
    h_attn = _attn_tail_kernel(h, o, p['Wo'], p['bo'], p['bn2_g'], p['bn2_b'])
    return _ffn_tail_kernel(h_local, h_attn, p['ff_W1'], p['ff_b1'],
                            p['ff_W2'], p['ff_b2'], p['bn3_g'], p['bn3_b'])


@jax.jit
def _forward_impl(x, edge_index, params):
    src = edge_index[0]
    dst = edge_index[1]
    pad = EPAD - E
    srcp = jnp.concatenate([src, jnp.zeros((pad,), jnp.int32)])
    dstp = jnp.concatenate([dst, jnp.full((pad,), N, jnp.int32)])
    srcp = srcp.reshape(EPAD // CHUNK, CHUNK)
    los = jnp.array([0, HALF], jnp.int32)[:, None]
    dl = dstp[None, :] - los
    dstloc = jnp.where((dl >= 0) & (dl < HALF), dl, HALF).astype(jnp.int32)
    dstloc = dstloc.reshape(SC_CORES, EPAD // CHUNK, CHUNK)

    zeros_ah = jnp.zeros((AGG_ROWS_PER_SUB, H), _F32)
    degp_raw = _sc_edge_count(dstloc, jnp.ones((CHUNK, H), _F32), zeros_ah)
    degp = jnp.concatenate(
        [degp_raw[0, :HALF, :], degp_raw[1, :N - HALF, :]], axis=0)
    h = _mm_bias(x, params['pre_W'], params['pre_b'], relu=True)
    h = _gps_layer(h, srcp, dstloc, degp, params['l1'])
    h = _gps_layer(h, srcp, dstloc, degp, params['l2'])
    return _head_kernel(h, params['head_W1'], params['head_b1'],
                        params['head_W2'], params['head_b2'])


def kernel(x, edge_index, params):
    return _forward_impl(x, edge_index, params)
